# baseline (speedup 1.0000x reference)
import jax
import jax.numpy as jnp
from jax import lax
from jax.experimental import pallas as pl
from jax.experimental.pallas import tpu as pltpu
from jax.experimental.pallas import tpu_sc as plsc

N = 100000
G = 64
L = 16
NW = 32
CHUNK = 3200
TBASE = (NW - 1) * CHUNK
TREAL = N - TBASE
TPAD = 896


def _sc_body(e_hbm, b_hbm, mt_hbm, at_hbm, sh_hbm, sc_hbm, out_hbm,
             e_v, b_v, at_v, o_v, mt_v, sh_t, sc_t, sem):
    cid = lax.axis_index("c")
    sid = lax.axis_index("s")
    wid = sid * 2 + cid
    base = wid * CHUNK
    is_last = wid == NW - 1

    tab_cps = (
        pltpu.async_copy(mt_hbm, mt_v, sem),
        pltpu.async_copy(sh_hbm, sh_t, sem),
        pltpu.async_copy(sc_hbm, sc_t, sem),
    )

    def gathers(b, a):
        m = plsc.load_gather(mt_v, [b])
        sh = plsc.load_gather(sh_t, [m, a])
        sc = plsc.load_gather(sc_t, [m, a])
        return sh, sc

    @pl.when(jnp.logical_not(is_last))
    def _main_path():
        cps = (
            pltpu.async_copy(e_hbm.at[0, pl.ds(base, CHUNK)], e_v, sem),
            pltpu.async_copy(b_hbm.at[pl.ds(base, CHUNK)],
                             b_v.at[pl.ds(0, CHUNK)], sem),
            pltpu.async_copy(at_hbm.at[pl.ds(base, CHUNK)],
                             at_v.at[pl.ds(0, CHUNK)], sem),
        )
        for cp in tab_cps + cps:
            cp.wait()

        @plsc.parallel_loop(0, CHUNK // L, 1, unroll=20)
        def _l(i):
            off = i * L
            sh, sc = gathers(b_v[pl.ds(off, L)], at_v[pl.ds(off, L)])
            o_v[pl.ds(off, L)] = e_v[pl.ds(off, L)] * sc + sh

        pltpu.sync_copy(o_v, out_hbm.at[0, pl.ds(base, CHUNK)])

    @pl.when(is_last)
    def _tail_path():
        cps = (
            pltpu.async_copy(e_hbm.at[0, pl.ds(pl.multiple_of(TBASE, 128),
                                               TPAD)],
                             e_v.at[pl.ds(0, TPAD)], sem),
            pltpu.async_copy(b_hbm.at[pl.ds(TBASE, TREAL)],
                             b_v.at[pl.ds(0, TREAL)], sem),
            pltpu.async_copy(at_hbm.at[pl.ds(TBASE, TREAL)],
                             at_v.at[pl.ds(0, TREAL)], sem),
        )
        for cp in tab_cps + cps:
            cp.wait()

        @plsc.parallel_loop(0, TPAD // L, 1, unroll=7)
        def _l(i):
            off = i * L
            b = b_v[pl.ds(off, L)] & (G - 1)
            a = at_v[pl.ds(off, L)] & 15
            sh, sc = gathers(b, a)
            o_v[pl.ds(off, L)] = e_v[pl.ds(off, L)] * sc + sh

        pltpu.sync_copy(o_v.at[pl.ds(0, TPAD)],
                        out_hbm.at[0, pl.ds(pl.multiple_of(TBASE, 128),
                                            TPAD)])


_mesh = plsc.VectorSubcoreMesh(core_axis_name="c", subcore_axis_name="s")

_sc_call = pl.kernel(
    _sc_body,
    mesh=_mesh,
    out_type=jax.ShapeDtypeStruct((1, N), jnp.float32),
    compiler_params=pltpu.CompilerParams(needs_layout_passes=False),
    scratch_types=[
        pltpu.VMEM((CHUNK,), jnp.float32),
        pltpu.VMEM((CHUNK,), jnp.int32),
        pltpu.VMEM((CHUNK,), jnp.int32),
        pltpu.VMEM((CHUNK,), jnp.float32),
        pltpu.VMEM((G,), jnp.int32),
        pltpu.VMEM((4, 16), jnp.float32),
        pltpu.VMEM((4, 16), jnp.float32),
        pltpu.SemaphoreType.DMA,
    ],
)


def kernel(scaled_atomic_energy, batch, modal_type, atom_type, shift, scale):
    e = scaled_atomic_energy.reshape(1, N)
    out = _sc_call(e, batch, modal_type, atom_type, shift, scale)
    return out.reshape(N, 1)


# speedup vs baseline: 1.0134x; 1.0134x over previous
import jax
import jax.numpy as jnp
from jax import lax
from jax.experimental import pallas as pl
from jax.experimental.pallas import tpu as pltpu
from jax.experimental.pallas import tpu_sc as plsc

N = 100000
G = 64
L = 16
NW = 32
CHUNK = 3200
TBASE = (NW - 1) * CHUNK
TREAL = N - TBASE
TPAD = 896


def _sc_body(e_hbm, b_hbm, mt_hbm, at_hbm, sh_hbm, sc_hbm, out_hbm,
             e_v, b_v, at_v, o_v, mt_v, sh_t, sc_t, sem):
    cid = lax.axis_index("c")
    sid = lax.axis_index("s")
    wid = sid * 2 + cid
    base = wid * CHUNK
    is_last = wid == NW - 1

    tab_cps = (
        pltpu.async_copy(mt_hbm, mt_v, sem),
        pltpu.async_copy(sh_hbm, sh_t, sem),
        pltpu.async_copy(sc_hbm, sc_t, sem),
    )

    def gathers(b, a):
        m = plsc.load_gather(mt_v, [b])
        sh = plsc.load_gather(sh_t, [m, a])
        sc = plsc.load_gather(sc_t, [m, a])
        return sh, sc

    @pl.when(jnp.logical_not(is_last))
    def _main_path():
        cps = (
            pltpu.async_copy(e_hbm.at[0, pl.ds(base, CHUNK)], e_v, sem),
            pltpu.async_copy(b_hbm.at[pl.ds(base, CHUNK)],
                             b_v.at[pl.ds(0, CHUNK)], sem),
            pltpu.async_copy(at_hbm.at[pl.ds(base, CHUNK)],
                             at_v.at[pl.ds(0, CHUNK)], sem),
        )
        for cp in tab_cps + cps:
            cp.wait()

        @plsc.parallel_loop(0, CHUNK // L, 1, unroll=10)
        def _l(i):
            off = i * L
            sh, sc = gathers(b_v[pl.ds(off, L)], at_v[pl.ds(off, L)])
            o_v[pl.ds(off, L)] = e_v[pl.ds(off, L)] * sc + sh

        pltpu.sync_copy(o_v, out_hbm.at[0, pl.ds(base, CHUNK)])

    @pl.when(is_last)
    def _tail_path():
        cps = (
            pltpu.async_copy(e_hbm.at[0, pl.ds(pl.multiple_of(TBASE, 128),
                                               TPAD)],
                             e_v.at[pl.ds(0, TPAD)], sem),
            pltpu.async_copy(b_hbm.at[pl.ds(TBASE, TREAL)],
                             b_v.at[pl.ds(0, TREAL)], sem),
            pltpu.async_copy(at_hbm.at[pl.ds(TBASE, TREAL)],
                             at_v.at[pl.ds(0, TREAL)], sem),
        )
        for cp in tab_cps + cps:
            cp.wait()

        @plsc.parallel_loop(0, TPAD // L, 1, unroll=7)
        def _l(i):
            off = i * L
            b = b_v[pl.ds(off, L)] & (G - 1)
            a = at_v[pl.ds(off, L)] & 15
            sh, sc = gathers(b, a)
            o_v[pl.ds(off, L)] = e_v[pl.ds(off, L)] * sc + sh

        pltpu.sync_copy(o_v.at[pl.ds(0, TPAD)],
                        out_hbm.at[0, pl.ds(pl.multiple_of(TBASE, 128),
                                            TPAD)])


_mesh = plsc.VectorSubcoreMesh(core_axis_name="c", subcore_axis_name="s")

_sc_call = pl.kernel(
    _sc_body,
    mesh=_mesh,
    out_type=jax.ShapeDtypeStruct((1, N), jnp.float32),
    compiler_params=pltpu.CompilerParams(needs_layout_passes=False),
    scratch_types=[
        pltpu.VMEM((CHUNK,), jnp.float32),
        pltpu.VMEM((CHUNK,), jnp.int32),
        pltpu.VMEM((CHUNK,), jnp.int32),
        pltpu.VMEM((CHUNK,), jnp.float32),
        pltpu.VMEM((G,), jnp.int32),
        pltpu.VMEM((4, 16), jnp.float32),
        pltpu.VMEM((4, 16), jnp.float32),
        pltpu.SemaphoreType.DMA,
    ],
)


def kernel(scaled_atomic_energy, batch, modal_type, atom_type, shift, scale):
    e = scaled_atomic_energy.reshape(1, N)
    out = _sc_call(e, batch, modal_type, atom_type, shift, scale)
    return out.reshape(N, 1)
